# baseline (speedup 1.0000x reference)
import functools
import math

import jax
import jax.numpy as jnp
from jax import lax
from jax.experimental import pallas as pl
from jax.experimental.pallas import tpu as pltpu
from jax.experimental.pallas import tpu_sc as plsc

_B, _C, _H, _W = 16, 96, 224, 224
_HW = _H * _W
_NSC = 4
_CH = _HW // _NSC
_CPG = 4
_NCG = 48 // _CPG
_VPC = _CH // 16
_K_INACTIVE = math.ceil(0.5 * _C)


def _sc_pool(x_hbm, m_hbm, pp_out, act_out, xb, mb, acc, sems):
    w = lax.axis_index("s") * 2 + lax.axis_index("c")
    b = w // 2
    half = w % 2
    ch0 = half * 48

    for ci in range(48):
        acc[ci] = jnp.zeros((16,), jnp.float32)

    macc = jnp.zeros((16,), jnp.float32)

    def xcopy(sc_i, cg, slot):
        return [
            pltpu.make_async_copy(
                x_hbm.at[b, ch0 + cg * _CPG + j, pl.ds(sc_i * _CH, _CH)],
                xb.at[slot, j],
                sems.at[slot, j])
            for j in range(_CPG)
        ]

    for sc_i in range(_NSC):
        pltpu.sync_copy(m_hbm.at[b, pl.ds(sc_i * _CH, _CH)], mb)

        def msum(k, mv):
            return mv + mb[pl.ds(k * 16, 16)]

        macc = lax.fori_loop(0, _VPC, msum, macc)

        for c in xcopy(sc_i, 0, 0):
            c.start()
        for cg in range(_NCG):
            slot = cg % 2
            if cg + 1 < _NCG:
                for c in xcopy(sc_i, cg + 1, 1 - slot):
                    c.start()
            for c in xcopy(sc_i, cg, slot):
                c.wait()

            def body(k, carry):
                a0, a1, a2, a3 = carry
                off = k * 16
                mv = mb[pl.ds(off, 16)]
                a0 = a0 + xb[slot, 0, pl.ds(off, 16)] * mv
                a1 = a1 + xb[slot, 1, pl.ds(off, 16)] * mv
                a2 = a2 + xb[slot, 2, pl.ds(off, 16)] * mv
                a3 = a3 + xb[slot, 3, pl.ds(off, 16)] * mv
                return a0, a1, a2, a3

            z = jnp.zeros((16,), jnp.float32)
            a0, a1, a2, a3 = lax.fori_loop(0, _VPC, body, (z, z, z, z))
            for j, aj in enumerate((a0, a1, a2, a3)):
                ci = cg * _CPG + j
                acc[ci] = acc[ci] + aj

    pltpu.sync_copy(acc, pp_out.at[b, pl.ds(ch0, 48)])

    @pl.when(half == 0)
    def _store_mask_sum():
        mb[pl.ds(0, 16)] = macc
        pltpu.sync_copy(mb.at[pl.ds(0, 16)], act_out.at[b, 0])


def _gate_kernel(pp_ref, act_ref, lasso_ref, w_ref, b_ref,
                 out_ref, lasso_out_ref, sacc_ref):
    i = pl.program_id(0)

    active = jnp.sum(act_ref[...])
    acc = pp_ref[0]

    @pl.when(i == 0)
    def _init_lasso_acc():
        sacc_ref[0] = 0.0

    ii = jax.lax.broadcasted_iota(jnp.int32, (_C, _C), 0)
    jj = jax.lax.broadcasted_iota(jnp.int32, (_C, _C), 1)
    eye = (ii == jj).astype(jnp.float32)
    pooled_col = jnp.sum(acc, axis=1, keepdims=True) / active
    pooled_row = jax.lax.dot_general(
        pooled_col, eye, (((0,), (0,)), ((), ())),
        preferred_element_type=jnp.float32,
        precision=jax.lax.Precision.HIGHEST)
    logits = jax.lax.dot_general(
        pooled_row, w_ref[...], (((1,), (1,)), ((), ())),
        preferred_element_type=jnp.float32,
        precision=jax.lax.Precision.DEFAULT)
    s_row = jax.nn.sigmoid(logits + b_ref[...])
    s_col = jax.lax.dot_general(
        eye, s_row, (((1,), (1,)), ((), ())),
        preferred_element_type=jnp.float32,
        precision=jax.lax.Precision.HIGHEST)
    s_r = jnp.broadcast_to(s_col, (_C, _C))
    s_c = jnp.broadcast_to(s_row, (_C, _C))
    beats = (s_r < s_c) | ((s_r == s_c) & (ii < jj))
    rank = jnp.sum(beats.astype(jnp.int32), axis=0, keepdims=True)
    out_ref[pl.ds(i, 1), :] = (rank >= _K_INACTIVE).astype(jnp.int32)
    sacc_ref[0] += jnp.sum(s_row)

    @pl.when(i == _B - 1)
    def _final():
        lasso_out_ref[0, 0] = lasso_ref[0, 0] + sacc_ref[0] / _B


def kernel(x, masked_feat, lasso_sum, W, b):
    xr = x.reshape(_B, _C, _HW)
    mr = masked_feat.reshape(_B, _HW)
    lr = lasso_sum.reshape(1, 1)
    br = b.reshape(1, _C)

    sc_pool = functools.partial(
        pl.kernel,
        mesh=plsc.VectorSubcoreMesh(core_axis_name="c", subcore_axis_name="s"),
        out_type=[
            jax.ShapeDtypeStruct((_B, _C, 16), jnp.float32),
            jax.ShapeDtypeStruct((_B, 1, 16), jnp.float32),
        ],
        scratch_types=[
            pltpu.VMEM((2, _CPG, _CH), jnp.float32),
            pltpu.VMEM((_CH,), jnp.float32),
            pltpu.VMEM((48, 16), jnp.float32),
            pltpu.SemaphoreType.DMA((2, _CPG)),
        ],
    )(_sc_pool)
    pp, act = sc_pool(xr, mr)

    out, lasso = pl.pallas_call(
        _gate_kernel,
        grid=(_B,),
        in_specs=[
            pl.BlockSpec((1, _C, 16), lambda i: (i, 0, 0)),
            pl.BlockSpec((1, 1, 16), lambda i: (i, 0, 0)),
            pl.BlockSpec(memory_space=pltpu.SMEM),
            pl.BlockSpec((_C, _C), lambda i: (0, 0)),
            pl.BlockSpec((1, _C), lambda i: (0, 0)),
        ],
        out_specs=[
            pl.BlockSpec((_B, _C), lambda i: (0, 0)),
            pl.BlockSpec(memory_space=pltpu.SMEM),
        ],
        out_shape=[
            jax.ShapeDtypeStruct((_B, _C), jnp.int32),
            jax.ShapeDtypeStruct((1, 1), jnp.float32),
        ],
        scratch_shapes=[
            pltpu.SMEM((1,), jnp.float32),
        ],
    )(pp, act, lr, W, br)
    return out, lasso.reshape(())


# speedup vs baseline: 1.0469x; 1.0469x over previous
import functools
import math

import jax
import jax.numpy as jnp
from jax import lax
from jax.experimental import pallas as pl
from jax.experimental.pallas import tpu as pltpu
from jax.experimental.pallas import tpu_sc as plsc

_B, _C, _H, _W = 16, 96, 224, 224
_HW = _H * _W
_NSC = 4
_CH = _HW // _NSC
_CPG = 4
_NCG = 48 // _CPG
_VPC = _CH // 16
_K_INACTIVE = math.ceil(0.5 * _C)


def _sc_pool(x_hbm, m_hbm, pp_out, act_out, xb, mb, acc, sems):
    w = lax.axis_index("s") * 2 + lax.axis_index("c")
    b = w // 2
    half = w % 2
    ch0 = half * 48

    for ci in range(48):
        acc[ci] = jnp.zeros((16,), jnp.float32)

    macc = jnp.zeros((16,), jnp.float32)

    def xcopy(sc_i, cg, slot):
        return [
            pltpu.make_async_copy(
                x_hbm.at[b, ch0 + cg * _CPG + j, pl.ds(sc_i * _CH, _CH)],
                xb.at[slot, j],
                sems.at[slot, j])
            for j in range(_CPG)
        ]

    for sc_i in range(_NSC):
        pltpu.sync_copy(m_hbm.at[b, pl.ds(sc_i * _CH, _CH)], mb)

        def msum(k, mv):
            for u in range(4):
                mv = mv + mb[pl.ds(k * 64 + u * 16, 16)]
            return mv

        macc = lax.fori_loop(0, _VPC // 4, msum, macc)

        for c in xcopy(sc_i, 0, 0):
            c.start()
        for cg in range(_NCG):
            slot = cg % 2
            if cg + 1 < _NCG:
                for c in xcopy(sc_i, cg + 1, 1 - slot):
                    c.start()
            for c in xcopy(sc_i, cg, slot):
                c.wait()

            def body(k, carry):
                a0, a1, a2, a3 = carry
                base = k * 64
                for u in range(4):
                    off = base + u * 16
                    mv = mb[pl.ds(off, 16)]
                    a0 = a0 + xb[slot, 0, pl.ds(off, 16)] * mv
                    a1 = a1 + xb[slot, 1, pl.ds(off, 16)] * mv
                    a2 = a2 + xb[slot, 2, pl.ds(off, 16)] * mv
                    a3 = a3 + xb[slot, 3, pl.ds(off, 16)] * mv
                return a0, a1, a2, a3

            z = jnp.zeros((16,), jnp.float32)
            a0, a1, a2, a3 = lax.fori_loop(0, _VPC // 4, body, (z, z, z, z))
            for j, aj in enumerate((a0, a1, a2, a3)):
                ci = cg * _CPG + j
                acc[ci] = acc[ci] + aj

    pltpu.sync_copy(acc, pp_out.at[b, pl.ds(ch0, 48)])

    @pl.when(half == 0)
    def _store_mask_sum():
        mb[pl.ds(0, 16)] = macc
        pltpu.sync_copy(mb.at[pl.ds(0, 16)], act_out.at[b, 0])


def _gate_kernel(pp_ref, act_ref, lasso_ref, w_ref, b_ref,
                 out_ref, lasso_out_ref, sacc_ref):
    i = pl.program_id(0)

    active = jnp.sum(act_ref[...])
    acc = pp_ref[0]

    @pl.when(i == 0)
    def _init_lasso_acc():
        sacc_ref[0] = 0.0

    ii = jax.lax.broadcasted_iota(jnp.int32, (_C, _C), 0)
    jj = jax.lax.broadcasted_iota(jnp.int32, (_C, _C), 1)
    eye = (ii == jj).astype(jnp.float32)
    pooled_col = jnp.sum(acc, axis=1, keepdims=True) / active
    pooled_row = jax.lax.dot_general(
        pooled_col, eye, (((0,), (0,)), ((), ())),
        preferred_element_type=jnp.float32,
        precision=jax.lax.Precision.HIGHEST)
    logits = jax.lax.dot_general(
        pooled_row, w_ref[...], (((1,), (1,)), ((), ())),
        preferred_element_type=jnp.float32,
        precision=jax.lax.Precision.DEFAULT)
    s_row = jax.nn.sigmoid(logits + b_ref[...])
    s_col = jax.lax.dot_general(
        eye, s_row, (((1,), (1,)), ((), ())),
        preferred_element_type=jnp.float32,
        precision=jax.lax.Precision.HIGHEST)
    s_r = jnp.broadcast_to(s_col, (_C, _C))
    s_c = jnp.broadcast_to(s_row, (_C, _C))
    beats = (s_r < s_c) | ((s_r == s_c) & (ii < jj))
    rank = jnp.sum(beats.astype(jnp.int32), axis=0, keepdims=True)
    out_ref[pl.ds(i, 1), :] = (rank >= _K_INACTIVE).astype(jnp.int32)
    sacc_ref[0] += jnp.sum(s_row)

    @pl.when(i == _B - 1)
    def _final():
        lasso_out_ref[0, 0] = lasso_ref[0, 0] + sacc_ref[0] / _B


def kernel(x, masked_feat, lasso_sum, W, b):
    xr = x.reshape(_B, _C, _HW)
    mr = masked_feat.reshape(_B, _HW)
    lr = lasso_sum.reshape(1, 1)
    br = b.reshape(1, _C)

    sc_pool = functools.partial(
        pl.kernel,
        mesh=plsc.VectorSubcoreMesh(core_axis_name="c", subcore_axis_name="s"),
        out_type=[
            jax.ShapeDtypeStruct((_B, _C, 16), jnp.float32),
            jax.ShapeDtypeStruct((_B, 1, 16), jnp.float32),
        ],
        scratch_types=[
            pltpu.VMEM((2, _CPG, _CH), jnp.float32),
            pltpu.VMEM((_CH,), jnp.float32),
            pltpu.VMEM((48, 16), jnp.float32),
            pltpu.SemaphoreType.DMA((2, _CPG)),
        ],
    )(_sc_pool)
    pp, act = sc_pool(xr, mr)

    out, lasso = pl.pallas_call(
        _gate_kernel,
        grid=(_B,),
        in_specs=[
            pl.BlockSpec((1, _C, 16), lambda i: (i, 0, 0)),
            pl.BlockSpec((1, 1, 16), lambda i: (i, 0, 0)),
            pl.BlockSpec(memory_space=pltpu.SMEM),
            pl.BlockSpec((_C, _C), lambda i: (0, 0)),
            pl.BlockSpec((1, _C), lambda i: (0, 0)),
        ],
        out_specs=[
            pl.BlockSpec((_B, _C), lambda i: (0, 0)),
            pl.BlockSpec(memory_space=pltpu.SMEM),
        ],
        out_shape=[
            jax.ShapeDtypeStruct((_B, _C), jnp.int32),
            jax.ShapeDtypeStruct((1, 1), jnp.float32),
        ],
        scratch_shapes=[
            pltpu.SMEM((1,), jnp.float32),
        ],
    )(pp, act, lr, W, br)
    return out, lasso.reshape(())


# speedup vs baseline: 1.2582x; 1.2018x over previous
import functools
import math

import jax
import jax.numpy as jnp
from jax import lax
from jax.experimental import pallas as pl
from jax.experimental.pallas import tpu as pltpu
from jax.experimental.pallas import tpu_sc as plsc

_B, _C, _H, _W = 16, 96, 224, 224
_HW = _H * _W
_BSC = 8
_BTC = _B - _BSC
_NSC = 4
_CH = _HW // _NSC
_CPG = 4
_CSUB = 24
_NCG = _CSUB // _CPG
_VPC = _CH // 16
_NS = 4
_CG = _C // _NS
_K_INACTIVE = math.ceil(0.5 * _C)


def _sc_pool(x_hbm, m_hbm, pp_out, act_out, xb, mb, acc, sems):
    w = lax.axis_index("s") * 2 + lax.axis_index("c")
    b = w // 4
    quarter = w % 4
    ch0 = quarter * _CSUB

    for ci in range(_CSUB):
        acc[ci] = jnp.zeros((16,), jnp.float32)

    macc = jnp.zeros((16,), jnp.float32)

    def xcopy(sc_i, cg, slot):
        return [
            pltpu.make_async_copy(
                x_hbm.at[b, ch0 + cg * _CPG + j, pl.ds(sc_i * _CH, _CH)],
                xb.at[slot, j],
                sems.at[slot, j])
            for j in range(_CPG)
        ]

    for sc_i in range(_NSC):
        pltpu.sync_copy(m_hbm.at[b, pl.ds(sc_i * _CH, _CH)], mb)

        def msum(k, mv):
            for u in range(4):
                mv = mv + mb[pl.ds(k * 64 + u * 16, 16)]
            return mv

        macc = lax.fori_loop(0, _VPC // 4, msum, macc)

        for c in xcopy(sc_i, 0, 0):
            c.start()
        for cg in range(_NCG):
            slot = cg % 2
            if cg + 1 < _NCG:
                for c in xcopy(sc_i, cg + 1, 1 - slot):
                    c.start()
            for c in xcopy(sc_i, cg, slot):
                c.wait()

            def body(k, carry):
                a0, a1, a2, a3 = carry
                base = k * 64
                for u in range(4):
                    off = base + u * 16
                    mv = mb[pl.ds(off, 16)]
                    a0 = a0 + xb[slot, 0, pl.ds(off, 16)] * mv
                    a1 = a1 + xb[slot, 1, pl.ds(off, 16)] * mv
                    a2 = a2 + xb[slot, 2, pl.ds(off, 16)] * mv
                    a3 = a3 + xb[slot, 3, pl.ds(off, 16)] * mv
                return a0, a1, a2, a3

            z = jnp.zeros((16,), jnp.float32)
            a0, a1, a2, a3 = lax.fori_loop(0, _VPC // 4, body, (z, z, z, z))
            for j, aj in enumerate((a0, a1, a2, a3)):
                ci = cg * _CPG + j
                acc[ci] = acc[ci] + aj

    pltpu.sync_copy(acc, pp_out.at[b, pl.ds(ch0, _CSUB)])

    @pl.when(quarter == 0)
    def _store_mask_sum():
        mb[pl.ds(0, 16)] = macc
        pltpu.sync_copy(mb.at[pl.ds(0, 16)], act_out.at[b, 0])


def _tc_pool(x0_ref, x1_ref, x2_ref, x3_ref, m_ref,
             pp_out_ref, act_out_ref, xbuf, sems):
    i = pl.program_id(0)
    x_refs = (x0_ref, x1_ref, x2_ref, x3_ref)

    def copies(slot, b):
        return [
            pltpu.make_async_copy(
                x_refs[s].at[_BSC + b, pl.ds(s * _CG, _CG), :],
                xbuf.at[slot, s],
                sems.at[slot, s])
            for s in range(_NS)
        ]

    @pl.when(i == 0)
    def _prime():
        for c in copies(0, 0):
            c.start()

    @pl.when(i + 1 < _BTC)
    def _prefetch():
        for c in copies((i + 1) % 2, i + 1):
            c.start()

    slot = i % 2
    for c in copies(slot, i):
        c.wait()

    m_row = m_ref[0]
    parts = []
    for s in range(_NS):
        xm = xbuf[slot, s] * m_row
        parts.append(jnp.sum(xm.reshape(_CG, _HW // 128, 128), axis=1))
    pp_out_ref[0] = jnp.concatenate(parts, axis=0)
    act_out_ref[i] = jnp.sum(m_row)


def _gate_kernel(pp_sc_ref, act_sc_ref, pp_tc_ref, act_tc_ref, lasso_ref,
                 w_ref, b_ref, out_ref, lasso_out_ref, sacc_ref):
    i = pl.program_id(0)
    is_sc = i < _BSC

    act_a = jnp.sum(act_sc_ref[...])
    act_b = act_tc_ref[jnp.maximum(i - _BSC, 0)]
    active = jnp.where(is_sc, act_a, act_b)
    pooled_sc = jnp.sum(pp_sc_ref[0], axis=1, keepdims=True)
    pooled_tc = jnp.sum(pp_tc_ref[0], axis=1, keepdims=True)

    @pl.when(i == 0)
    def _init_lasso_acc():
        sacc_ref[0] = 0.0

    ii = jax.lax.broadcasted_iota(jnp.int32, (_C, _C), 0)
    jj = jax.lax.broadcasted_iota(jnp.int32, (_C, _C), 1)
    eye = (ii == jj).astype(jnp.float32)
    pooled_col = jnp.where(is_sc, pooled_sc, pooled_tc) / active
    pooled_row = jax.lax.dot_general(
        pooled_col, eye, (((0,), (0,)), ((), ())),
        preferred_element_type=jnp.float32,
        precision=jax.lax.Precision.HIGHEST)
    logits = jax.lax.dot_general(
        pooled_row, w_ref[...], (((1,), (1,)), ((), ())),
        preferred_element_type=jnp.float32,
        precision=jax.lax.Precision.DEFAULT)
    s_row = jax.nn.sigmoid(logits + b_ref[...])
    s_col = jax.lax.dot_general(
        eye, s_row, (((1,), (1,)), ((), ())),
        preferred_element_type=jnp.float32,
        precision=jax.lax.Precision.HIGHEST)
    s_r = jnp.broadcast_to(s_col, (_C, _C))
    s_c = jnp.broadcast_to(s_row, (_C, _C))
    beats = (s_r < s_c) | ((s_r == s_c) & (ii < jj))
    rank = jnp.sum(beats.astype(jnp.int32), axis=0, keepdims=True)
    out_ref[pl.ds(i, 1), :] = (rank >= _K_INACTIVE).astype(jnp.int32)
    sacc_ref[0] += jnp.sum(s_row)

    @pl.when(i == _B - 1)
    def _final():
        lasso_out_ref[0, 0] = lasso_ref[0, 0] + sacc_ref[0] / _B


def kernel(x, masked_feat, lasso_sum, W, b):
    xr = x.reshape(_B, _C, _HW)
    mr = masked_feat.reshape(_B, _HW)
    mr3 = masked_feat.reshape(_B, 1, _HW)
    lr = lasso_sum.reshape(1, 1)
    br = b.reshape(1, _C)

    sc_pool = functools.partial(
        pl.kernel,
        mesh=plsc.VectorSubcoreMesh(core_axis_name="c", subcore_axis_name="s"),
        out_type=[
            jax.ShapeDtypeStruct((_BSC, _C, 16), jnp.float32),
            jax.ShapeDtypeStruct((_BSC, 1, 16), jnp.float32),
        ],
        scratch_types=[
            pltpu.VMEM((2, _CPG, _CH), jnp.float32),
            pltpu.VMEM((_CH,), jnp.float32),
            pltpu.VMEM((_CSUB, 16), jnp.float32),
            pltpu.SemaphoreType.DMA((2, _CPG)),
        ],
    )(_sc_pool)
    pp_sc, act_sc = sc_pool(xr, mr)

    pp_tc, act_tc = pl.pallas_call(
        _tc_pool,
        grid=(_BTC,),
        in_specs=[
            pl.BlockSpec(memory_space=pl.ANY),
            pl.BlockSpec(memory_space=pl.ANY),
            pl.BlockSpec(memory_space=pl.ANY),
            pl.BlockSpec(memory_space=pl.ANY),
            pl.BlockSpec((1, 1, _HW), lambda i: (_BSC + i, 0, 0)),
        ],
        out_specs=[
            pl.BlockSpec((1, _C, 128), lambda i: (i, 0, 0)),
            pl.BlockSpec(memory_space=pltpu.SMEM),
        ],
        out_shape=[
            jax.ShapeDtypeStruct((_BTC, _C, 128), jnp.float32),
            jax.ShapeDtypeStruct((_BTC,), jnp.float32),
        ],
        scratch_shapes=[
            pltpu.VMEM((2, _NS, _CG, _HW), jnp.float32),
            pltpu.SemaphoreType.DMA((2, _NS)),
        ],
    )(xr, xr, xr, xr, mr3)

    out, lasso = pl.pallas_call(
        _gate_kernel,
        grid=(_B,),
        in_specs=[
            pl.BlockSpec((1, _C, 16), lambda i: (jnp.minimum(i, _BSC - 1), 0, 0)),
            pl.BlockSpec((1, 1, 16), lambda i: (jnp.minimum(i, _BSC - 1), 0, 0)),
            pl.BlockSpec((1, _C, 128),
                         lambda i: (jnp.maximum(i - _BSC, 0), 0, 0)),
            pl.BlockSpec(memory_space=pltpu.SMEM),
            pl.BlockSpec(memory_space=pltpu.SMEM),
            pl.BlockSpec((_C, _C), lambda i: (0, 0)),
            pl.BlockSpec((1, _C), lambda i: (0, 0)),
        ],
        out_specs=[
            pl.BlockSpec((_B, _C), lambda i: (0, 0)),
            pl.BlockSpec(memory_space=pltpu.SMEM),
        ],
        out_shape=[
            jax.ShapeDtypeStruct((_B, _C), jnp.int32),
            jax.ShapeDtypeStruct((1, 1), jnp.float32),
        ],
        scratch_shapes=[
            pltpu.SMEM((1,), jnp.float32),
        ],
    )(pp_sc, act_sc, pp_tc, act_tc, lr, W, br)
    return out, lasso.reshape(())
